# baseline (speedup 1.0000x reference)
import jax
import jax.numpy as jnp
from jax import lax
from jax.experimental import pallas as pl
from jax.experimental.pallas import tpu as pltpu
from jax.experimental.pallas import tpu_sc as plsc

EMBED_DIM = 32
BATCH = 16384
NROWS = 1000000

NC = 2
NS = 16
NW = NC * NS
BPW = BATCH // NW
CHUNK = 128
NCHUNK = BPW // CHUNK

BLK = 32768
GRID = (NROWS + BLK - 1) // BLK


def _tc_body(w_ref, u0_ref, u1_ref, b0_ref, b1_ref, su_ref, sb_ref):
    w = w_ref[...]
    bias = w[0, 2 * EMBED_DIM]
    dn = (((1,), (0,)), ((), ()))

    def dot(lo, ref):
        return jax.lax.dot_general(w[:, lo:lo + 16], ref[...], dn,
                                   preferred_element_type=jnp.float32)

    su = dot(0, u0_ref) + dot(16, u1_ref)
    sb = dot(32, b0_ref) + dot(48, b1_ref)
    su_ref[...] = su.reshape(BLK) + bias
    sb_ref[...] = sb.reshape(BLK)


def _sc_body(users_hbm, books_hbm, su_hbm, sb_hbm, out_hbm,
             uidx_v, bidx_v, vals_u, vals_b, out_v, sem):
    wid = lax.axis_index("s") * NC + lax.axis_index("c")
    base = wid * BPW

    pltpu.sync_copy(users_hbm.at[wid], uidx_v)
    pltpu.sync_copy(books_hbm.at[wid], bidx_v)

    copies = []
    for j in range(NCHUNK):
        lo = j * CHUNK
        copies.append(pltpu.async_copy(
            su_hbm.at[uidx_v.at[pl.ds(lo, CHUNK)]], vals_u.at[pl.ds(lo, CHUNK)], sem))
        copies.append(pltpu.async_copy(
            sb_hbm.at[bidx_v.at[pl.ds(lo, CHUNK)]], vals_b.at[pl.ds(lo, CHUNK)], sem))
    for c in copies:
        c.wait()

    def group(g, carry):
        vu = vals_u[pl.ds(g * 16, 16)]
        vb = vals_b[pl.ds(g * 16, 16)]
        out_v[pl.ds(g * 16, 16)] = 5.0 / (1.0 + jnp.exp(-(vu + vb)))
        return carry

    lax.fori_loop(0, BPW // 16, group, 0)

    pltpu.sync_copy(out_v, out_hbm.at[pl.ds(base, BPW)])


@jax.jit
def _run(users_r, books_r, u0, u1, b0, b1, wrow):
    half = pl.BlockSpec((EMBED_DIM // 2, BLK), lambda j: (0, j))
    su, sb = pl.pallas_call(
        _tc_body,
        grid=(GRID,),
        in_specs=[
            pl.BlockSpec((1, 128), lambda j: (0, 0)),
            half, half, half, half,
        ],
        out_specs=[
            pl.BlockSpec((BLK,), lambda j: (j,)),
            pl.BlockSpec((BLK,), lambda j: (j,)),
        ],
        out_shape=[
            jax.ShapeDtypeStruct((NROWS,), jnp.float32),
            jax.ShapeDtypeStruct((NROWS,), jnp.float32),
        ],
    )(wrow, u0, u1, b0, b1)

    mesh = plsc.VectorSubcoreMesh(core_axis_name="c", subcore_axis_name="s")
    out = pl.kernel(
        _sc_body, mesh=mesh,
        out_type=jax.ShapeDtypeStruct((BATCH,), jnp.float32),
        scratch_types=[
            pltpu.VMEM((BPW,), jnp.int32),
            pltpu.VMEM((BPW,), jnp.int32),
            pltpu.VMEM((BPW,), jnp.float32),
            pltpu.VMEM((BPW,), jnp.float32),
            pltpu.VMEM((BPW,), jnp.float32),
            pltpu.SemaphoreType.DMA,
        ],
        compiler_params=pltpu.CompilerParams(
            needs_layout_passes=False,
        ),
    )(users_r, books_r, su, sb)
    return out


def kernel(users, books, user_table, book_table, W, b):
    users_r = users.astype(jnp.int32).reshape(NW, BPW)
    books_r = books.astype(jnp.int32).reshape(NW, BPW)
    wrow = jnp.zeros((1, 128), jnp.float32).at[0, :65].set(
        jnp.concatenate([W.reshape(-1), b]).astype(jnp.float32))
    ut_t = user_table.T
    bt_t = book_table.T
    out = _run(users_r, books_r, ut_t[:16], ut_t[16:], bt_t[:16], bt_t[16:],
               wrow)
    return out.reshape(BATCH, 1)


# speedup vs baseline: 2.5125x; 2.5125x over previous
import jax
import jax.numpy as jnp
from jax import lax
from jax.experimental import pallas as pl
from jax.experimental.pallas import tpu as pltpu
from jax.experimental.pallas import tpu_sc as plsc

EMBED_DIM = 32
BATCH = 16384
NROWS = 1000000

NC = 2
NS = 16
NW = NC * NS
BPW = BATCH // NW
CHUNK = 128
NCHUNK = BPW // CHUNK

BLK = 32768
GRID = (NROWS + BLK - 1) // BLK


def _tc_body(w_ref, u_ref, b_ref, su_ref, sb_ref):
    w = w_ref[...]
    bias = w[0, 2 * EMBED_DIM]
    dn = (((1,), (0,)), ((), ()))
    su = jax.lax.dot_general(w[:, :EMBED_DIM], u_ref[...], dn,
                             preferred_element_type=jnp.float32)
    sb = jax.lax.dot_general(w[:, EMBED_DIM:2 * EMBED_DIM], b_ref[...], dn,
                             preferred_element_type=jnp.float32)
    su_ref[...] = su.reshape(BLK) + bias
    sb_ref[...] = sb.reshape(BLK)


def _sc_body(users_hbm, books_hbm, su_hbm, sb_hbm, out_hbm,
             uidx_v, bidx_v, vals_u, vals_b, out_v, sem):
    wid = lax.axis_index("s") * NC + lax.axis_index("c")
    base = wid * BPW

    pltpu.sync_copy(users_hbm.at[wid], uidx_v)
    pltpu.sync_copy(books_hbm.at[wid], bidx_v)

    copies = []
    for j in range(NCHUNK):
        lo = j * CHUNK
        copies.append(pltpu.async_copy(
            su_hbm.at[uidx_v.at[pl.ds(lo, CHUNK)]], vals_u.at[pl.ds(lo, CHUNK)], sem))
        copies.append(pltpu.async_copy(
            sb_hbm.at[bidx_v.at[pl.ds(lo, CHUNK)]], vals_b.at[pl.ds(lo, CHUNK)], sem))
    for c in copies:
        c.wait()

    def group(g, carry):
        vu = vals_u[pl.ds(g * 16, 16)]
        vb = vals_b[pl.ds(g * 16, 16)]
        out_v[pl.ds(g * 16, 16)] = 5.0 / (1.0 + jnp.exp(-(vu + vb)))
        return carry

    lax.fori_loop(0, BPW // 16, group, 0)

    pltpu.sync_copy(out_v, out_hbm.at[pl.ds(base, BPW)])


@jax.jit
def _run(users_r, books_r, ut_t, bt_t, wrow):
    su, sb = pl.pallas_call(
        _tc_body,
        grid=(GRID,),
        in_specs=[
            pl.BlockSpec((1, 128), lambda j: (0, 0)),
            pl.BlockSpec((EMBED_DIM, BLK), lambda j: (0, j)),
            pl.BlockSpec((EMBED_DIM, BLK), lambda j: (0, j)),
        ],
        out_specs=[
            pl.BlockSpec((BLK,), lambda j: (j,)),
            pl.BlockSpec((BLK,), lambda j: (j,)),
        ],
        out_shape=[
            jax.ShapeDtypeStruct((NROWS,), jnp.float32),
            jax.ShapeDtypeStruct((NROWS,), jnp.float32),
        ],
    )(wrow, ut_t, bt_t)

    mesh = plsc.VectorSubcoreMesh(core_axis_name="c", subcore_axis_name="s")
    out = pl.kernel(
        _sc_body, mesh=mesh,
        out_type=jax.ShapeDtypeStruct((BATCH,), jnp.float32),
        scratch_types=[
            pltpu.VMEM((BPW,), jnp.int32),
            pltpu.VMEM((BPW,), jnp.int32),
            pltpu.VMEM((BPW,), jnp.float32),
            pltpu.VMEM((BPW,), jnp.float32),
            pltpu.VMEM((BPW,), jnp.float32),
            pltpu.SemaphoreType.DMA,
        ],
        compiler_params=pltpu.CompilerParams(
            needs_layout_passes=False,
        ),
    )(users_r, books_r, su, sb)
    return out


def kernel(users, books, user_table, book_table, W, b):
    users_r = users.astype(jnp.int32).reshape(NW, BPW)
    books_r = books.astype(jnp.int32).reshape(NW, BPW)
    wrow = jnp.zeros((1, 128), jnp.float32).at[0, :65].set(
        jnp.concatenate([W.reshape(-1), b]).astype(jnp.float32))
    out = _run(users_r, books_r, user_table.T, book_table.T, wrow)
    return out.reshape(BATCH, 1)
